# baseline (speedup 1.0000x reference)
import functools

import jax
import jax.numpy as jnp
from jax import lax
from jax.experimental import pallas as pl
from jax.experimental.pallas import tpu as pltpu
from jax.experimental.pallas import tpu_sc as plsc

EMBED_DIM = 128
SEQ = 50
NROWS = 4096
NUM_CORES = 2
NUM_SUBCORES = 16
NUM_WORKERS = NUM_CORES * NUM_SUBCORES
BLOCK = NROWS // NUM_WORKERS
STEPS = SEQ
NBUF = 7
LOOK = 5
_LAST_STEADY = (STEPS - LOOK - NBUF) // NBUF

_mesh = plsc.VectorSubcoreMesh(core_axis_name="c", subcore_axis_name="s")


@functools.partial(
    pl.kernel,
    out_type=jax.ShapeDtypeStruct((SEQ, NROWS, EMBED_DIM), jnp.float32),
    mesh=_mesh,
    scratch_types=[
        pltpu.VMEM((STEPS, BLOCK), jnp.int32),
        pltpu.VMEM((NBUF, BLOCK, EMBED_DIM), jnp.float32),
    ] + [pltpu.SemaphoreType.DMA] * (2 * NBUF),
)
def _embedding_gather(idx_hbm, table_hbm, out_hbm, idx_v, rows_v, *sems):
    gsem = sems[:NBUF]
    ssem = sems[NBUF:]
    wid = lax.axis_index("c") * NUM_SUBCORES + lax.axis_index("s")
    base = wid * BLOCK
    pltpu.sync_copy(idx_hbm.at[:, pl.ds(base, BLOCK)], idx_v)

    def start_gather(step, slot):
        pltpu.async_copy(table_hbm.at[idx_v.at[step]], rows_v.at[slot],
                         gsem[slot])

    def wait_gather(step, slot):
        pltpu.make_async_copy(table_hbm.at[idx_v.at[step]], rows_v.at[slot],
                              gsem[slot]).wait()

    def start_store(step, slot):
        pltpu.async_copy(rows_v.at[slot],
                         out_hbm.at[step].at[pl.ds(base, BLOCK)],
                         ssem[slot])

    def wait_store(step, slot):
        pltpu.make_async_copy(rows_v.at[slot],
                              out_hbm.at[step].at[pl.ds(base, BLOCK)],
                              ssem[slot]).wait()

    def step_body(t, b, prefetch, storewait):
        g = t + LOOK
        gs = (b + LOOK) % NBUF
        if prefetch:
            if storewait:
                wait_store(g - NBUF, gs)
            start_gather(g, gs)
        wait_gather(t, b)
        start_store(t, b)

    for t in range(LOOK):
        start_gather(t, t % NBUF)

    for b in range(NBUF):
        step_body(b, b, prefetch=True, storewait=(b + LOOK >= NBUF))

    def round_body(r, carry):
        for b in range(NBUF):
            step_body(r * NBUF + b, b, prefetch=True, storewait=True)
        return carry

    lax.fori_loop(1, _LAST_STEADY + 1, round_body, 0)

    for t in range((_LAST_STEADY + 1) * NBUF, STEPS):
        step_body(t, t % NBUF, prefetch=(t + LOOK < STEPS), storewait=True)

    for t in range(STEPS - NBUF, STEPS):
        wait_store(t, t % NBUF)


def kernel(token_ids, weight):
    idx_t = jnp.transpose(token_ids.astype(jnp.int32))
    out_t = _embedding_gather(idx_t, weight)
    return jnp.transpose(out_t, (1, 0, 2))


# speedup vs baseline: 1.0033x; 1.0033x over previous
import functools

import jax
import jax.numpy as jnp
from jax import lax
from jax.experimental import pallas as pl
from jax.experimental.pallas import tpu as pltpu
from jax.experimental.pallas import tpu_sc as plsc

EMBED_DIM = 128
SEQ = 50
NROWS = 4096
NUM_CORES = 2
NUM_SUBCORES = 16
NUM_WORKERS = NUM_CORES * NUM_SUBCORES
BLOCK = NROWS // NUM_WORKERS
STEPS = SEQ
NBUF = 7
LOOK = 4
_LAST_STEADY = (STEPS - LOOK - NBUF) // NBUF

_mesh = plsc.VectorSubcoreMesh(core_axis_name="c", subcore_axis_name="s")


@functools.partial(
    pl.kernel,
    out_type=jax.ShapeDtypeStruct((SEQ, NROWS, EMBED_DIM), jnp.float32),
    mesh=_mesh,
    scratch_types=[
        pltpu.VMEM((STEPS, BLOCK), jnp.int32),
        pltpu.VMEM((NBUF, BLOCK, EMBED_DIM), jnp.float32),
    ] + [pltpu.SemaphoreType.DMA] * (2 * NBUF),
)
def _embedding_gather(idx_hbm, table_hbm, out_hbm, idx_v, rows_v, *sems):
    gsem = sems[:NBUF]
    ssem = sems[NBUF:]
    wid = lax.axis_index("c") * NUM_SUBCORES + lax.axis_index("s")
    base = wid * BLOCK
    pltpu.sync_copy(idx_hbm.at[:, pl.ds(base, BLOCK)], idx_v)

    def start_gather(step, slot):
        pltpu.async_copy(table_hbm.at[idx_v.at[step]], rows_v.at[slot],
                         gsem[slot])

    def wait_gather(step, slot):
        pltpu.make_async_copy(table_hbm.at[idx_v.at[step]], rows_v.at[slot],
                              gsem[slot]).wait()

    def start_store(step, slot):
        pltpu.async_copy(rows_v.at[slot],
                         out_hbm.at[step].at[pl.ds(base, BLOCK)],
                         ssem[slot])

    def wait_store(step, slot):
        pltpu.make_async_copy(rows_v.at[slot],
                              out_hbm.at[step].at[pl.ds(base, BLOCK)],
                              ssem[slot]).wait()

    def step_body(t, b, prefetch, storewait):
        g = t + LOOK
        gs = (b + LOOK) % NBUF
        if prefetch:
            if storewait:
                wait_store(g - NBUF, gs)
            start_gather(g, gs)
        wait_gather(t, b)
        start_store(t, b)

    for t in range(LOOK):
        start_gather(t, t % NBUF)

    for b in range(NBUF):
        step_body(b, b, prefetch=True, storewait=(b + LOOK >= NBUF))

    def round_body(r, carry):
        for b in range(NBUF):
            step_body(r * NBUF + b, b, prefetch=True, storewait=True)
        return carry

    lax.fori_loop(1, _LAST_STEADY + 1, round_body, 0)

    for t in range((_LAST_STEADY + 1) * NBUF, STEPS):
        step_body(t, t % NBUF, prefetch=(t + LOOK < STEPS), storewait=True)

    for t in range(STEPS - NBUF, STEPS):
        wait_store(t, t % NBUF)


def kernel(token_ids, weight):
    idx_t = jnp.transpose(token_ids.astype(jnp.int32))
    out_t = _embedding_gather(idx_t, weight)
    return jnp.transpose(out_t, (1, 0, 2))
